# Initial kernel scaffold; baseline (speedup 1.0000x reference)
import functools

import jax
import jax.numpy as jnp
from jax import lax
from jax.experimental import pallas as pl
from jax.experimental.pallas import tpu as pltpu
from jax.experimental.pallas import tpu_sc as plsc

NC = 2
NS = 16
NW = NC * NS
C = 80


def _sc_body(N, E, u_hbm, src_hbm, dst_hbm, w_hbm, out_hbm,
             idx_s, idx_t, us, ut, wv, ws, wt, zv, av, ov, b_sh,
             sem1, sem2):
    epw = E // NW
    nchunk = epw // C
    rpt = N // NS
    rb = rpt // 5
    cid = lax.axis_index("c")
    sid = lax.axis_index("s")
    wid = sid * NC + cid

    zero16 = jnp.zeros((16,), jnp.float32)

    def zrow(r, carry):
        for g in range(4):
            zv[r, pl.ds(g * 16, 16)] = zero16
        return carry

    lax.fori_loop(0, rb, zrow, 0)
    row0 = sid * rpt
    for j in range(5):
        pltpu.sync_copy(zv, b_sh.at[pl.ds(row0 + j * rb, rb)])
    plsc.subcore_barrier()

    def chunk_body(i, carry):
        base = wid * epw + i * C
        pltpu.sync_copy(src_hbm.at[pl.ds(base, C)], idx_s)
        pltpu.sync_copy(dst_hbm.at[pl.ds(base, C)], idx_t)
        cp1 = pltpu.async_copy(u_hbm.at[idx_s], us, sem1)
        cp2 = pltpu.async_copy(u_hbm.at[idx_t], ut, sem2)
        pltpu.sync_copy(w_hbm.at[pl.ds(base, C)], wv)
        cp1.wait()
        cp2.wait()

        def edge_body(e, ecarry):
            acc_a = jnp.zeros((16,), jnp.float32)
            acc_b = jnp.zeros((16,), jnp.float32)
            acc_c = jnp.zeros((16,), jnp.float32)
            for k in range(8):
                xs = us[e, pl.ds(k * 16, 16)]
                xt = ut[e, pl.ds(k * 16, 16)]
                vs = us[e, pl.ds(128 + k * 16, 16)]
                vt = ut[e, pl.ds(128 + k * 16, 16)]
                d = xt - xs
                acc_a = acc_a + d * d
                acc_b = acc_b + vs * d
                acc_c = acc_c + vt * d
            inv = 1.0 / jnp.maximum(jnp.sum(acc_a), 1e-6)
            cs = jnp.sum(acc_b) * inv
            ct = jnp.sum(acc_c) * inv
            for g in range(4):
                wvec = wv[e, pl.ds(g * 16, 16)]
                ws[e, pl.ds(g * 16, 16)] = wvec * cs
                wt[e, pl.ds(g * 16, 16)] = wvec * ct
            return ecarry

        lax.fori_loop(0, C, edge_body, 0)
        pltpu.sync_copy(ws, b_sh.at[idx_s], add=True)
        pltpu.sync_copy(wt, b_sh.at[idx_t], add=True)
        return carry

    lax.fori_loop(0, nchunk, chunk_body, 0)
    plsc.subcore_barrier()

    iota16 = lax.iota(jnp.int32, 16)
    perms = []
    for g in range(4):
        fl = iota16 + (g * 16)
        i8 = fl // 8
        j8 = fl % 8
        perms.append(j8 * 8 + i8)

    for j in range(5):
        pltpu.sync_copy(b_sh.at[pl.ds(row0 + j * rb, rb)], av)

        def arow(r, carry):
            rsplat = jnp.full((16,), r, jnp.int32)
            for g in range(4):
                bvec = av[r, pl.ds(g * 16, 16)]
                btv = plsc.load_gather(av, [rsplat, perms[g]])
                ov[r, pl.ds(g * 16, 16)] = 0.5 * (bvec - btv)
            return carry

        lax.fori_loop(0, rb, arow, 0)
        pltpu.sync_copy(
            ov, out_hbm.at[pl.ds(cid * N + row0 + j * rb, rb)])


@functools.lru_cache(maxsize=None)
def _make_sc(N, E):
    body = functools.partial(_sc_body, N, E)
    return pl.kernel(
        body,
        out_type=jax.ShapeDtypeStruct((NC * N, 64), jnp.float32),
        mesh=plsc.VectorSubcoreMesh(
            core_axis_name="c", subcore_axis_name="s",
            num_cores=NC, num_subcores=NS),
        scratch_types=[
            pltpu.VMEM((C,), jnp.int32),
            pltpu.VMEM((C,), jnp.int32),
            pltpu.VMEM((C, 256), jnp.float32),
            pltpu.VMEM((C, 256), jnp.float32),
            pltpu.VMEM((C, 64), jnp.float32),
            pltpu.VMEM((C, 64), jnp.float32),
            pltpu.VMEM((C, 64), jnp.float32),
            pltpu.VMEM((125, 64), jnp.float32),
            pltpu.VMEM((125, 64), jnp.float32),
            pltpu.VMEM((125, 64), jnp.float32),
            pltpu.VMEM_SHARED((N, 64), jnp.float32),
            pltpu.SemaphoreType.DMA,
            pltpu.SemaphoreType.DMA,
        ],
    )


def _combine_body(p_ref, o_ref):
    o_ref[...] = p_ref[0] + p_ref[1]


@functools.lru_cache(maxsize=None)
def _make_combine(R):
    br = R // 5
    return pl.pallas_call(
        _combine_body,
        out_shape=jax.ShapeDtypeStruct((R, 128), jnp.float32),
        grid=(R // br,),
        in_specs=[pl.BlockSpec((2, br, 128), lambda i: (0, i, 0))],
        out_specs=pl.BlockSpec((br, 128), lambda i: (i, 0)),
    )


@jax.jit
def kernel(x, v, edges, omega_params):
    N, D = x.shape
    E = edges.shape[0]
    K = omega_params.shape[1]
    assert D == 128 and K == 8
    assert E % (NW * C) == 0 and N % (NS * 5) == 0

    u = jnp.concatenate([x, v], axis=1)
    src = edges[:, 0]
    dst = edges[:, 1]
    w = omega_params.reshape(E, K * K)

    partial = _make_sc(N, E)(u, src, dst, w)
    r = (N * K * K) // 128
    summed = _make_combine(r)(partial.reshape(2, r, 128))
    return summed.reshape(N, K, K)


# speedup vs baseline: 19.8728x; 19.8728x over previous
import functools

import jax
import jax.numpy as jnp
from jax import lax
from jax.experimental import pallas as pl
from jax.experimental.pallas import tpu as pltpu
from jax.experimental.pallas import tpu_sc as plsc

NC = 2
NS = 16
NW = NC * NS
C = 80


def _sc_body(N, E, u_hbm, src_hbm, dst_hbm, w_hbm, out_hbm,
             idx_s, idx_t, us, ut, wv, ws, wt, cs_arr, ct_arr,
             zv, av, ov, b_sh, sem1, sem2):
    epw = E // NW
    nchunk = epw // C
    rpt = N // NS
    rb = rpt // 5
    cid = lax.axis_index("c")
    sid = lax.axis_index("s")
    wid = sid * NC + cid

    zero16 = jnp.zeros((16,), jnp.float32)

    def zrow(r, carry):
        for g in range(4):
            zv[r, pl.ds(g * 16, 16)] = zero16
        return carry

    lax.fori_loop(0, rb, zrow, 0)
    row0 = sid * rpt
    for j in range(5):
        pltpu.sync_copy(zv, b_sh.at[pl.ds(row0 + j * rb, rb)])
    plsc.subcore_barrier()

    def chunk_body(i, carry):
        base = wid * epw + i * C
        pltpu.sync_copy(src_hbm.at[pl.ds(base, C)], idx_s)
        pltpu.sync_copy(dst_hbm.at[pl.ds(base, C)], idx_t)
        cp1 = pltpu.async_copy(u_hbm.at[idx_s], us, sem1)
        cp2 = pltpu.async_copy(u_hbm.at[idx_t], ut, sem2)
        pltpu.sync_copy(w_hbm.at[pl.ds(base, C)], wv)
        cp1.wait()
        cp2.wait()

        iota16 = lax.iota(jnp.int32, 16)
        zf = jnp.zeros((16,), jnp.float32)

        def group_body(gi, gcarry):
            rows = gi * 16 + iota16

            def d_body(d, accs):
                a, b, c = accs
                colx = jnp.full((16,), d, jnp.int32)
                colv = colx + 128
                xs = plsc.load_gather(us, [rows, colx])
                xt = plsc.load_gather(ut, [rows, colx])
                vs = plsc.load_gather(us, [rows, colv])
                vt = plsc.load_gather(ut, [rows, colv])
                dd = xt - xs
                return (a + dd * dd, b + vs * dd, c + vt * dd)

            a, b, c = lax.fori_loop(0, 128, d_body, (zf, zf, zf))
            inv = 1.0 / jnp.maximum(a, 1e-6)
            cs_arr[pl.ds(gi * 16, 16)] = b * inv
            ct_arr[pl.ds(gi * 16, 16)] = c * inv
            return gcarry

        lax.fori_loop(0, C // 16, group_body, 0)

        def edge_body(e, ecarry):
            esplat = jnp.full((16,), e, jnp.int32)
            csb = plsc.load_gather(cs_arr, [esplat])
            ctb = plsc.load_gather(ct_arr, [esplat])
            for g in range(4):
                wvec = wv[e, pl.ds(g * 16, 16)]
                ws[e, pl.ds(g * 16, 16)] = wvec * csb
                wt[e, pl.ds(g * 16, 16)] = wvec * ctb
            return ecarry

        lax.fori_loop(0, C, edge_body, 0)
        pltpu.sync_copy(ws, b_sh.at[idx_s], add=True)
        pltpu.sync_copy(wt, b_sh.at[idx_t], add=True)
        return carry

    lax.fori_loop(0, nchunk, chunk_body, 0)
    plsc.subcore_barrier()

    iota16 = lax.iota(jnp.int32, 16)
    perms = []
    for g in range(4):
        fl = iota16 + (g * 16)
        i8 = fl // 8
        j8 = fl % 8
        perms.append(j8 * 8 + i8)

    for j in range(5):
        pltpu.sync_copy(b_sh.at[pl.ds(row0 + j * rb, rb)], av)

        def arow(r, carry):
            rsplat = jnp.full((16,), r, jnp.int32)
            for g in range(4):
                bvec = av[r, pl.ds(g * 16, 16)]
                btv = plsc.load_gather(av, [rsplat, perms[g]])
                ov[pl.ds(r * 64 + g * 16, 16)] = 0.5 * (bvec - btv)
            return carry

        lax.fori_loop(0, rb, arow, 0)
        pltpu.sync_copy(
            ov,
            out_hbm.at[pl.ds((cid * N + row0 + j * rb) * 64, rb * 64)])


@functools.lru_cache(maxsize=None)
def _make_sc(N, E):
    body = functools.partial(_sc_body, N, E)
    return pl.kernel(
        body,
        out_type=jax.ShapeDtypeStruct((NC * N * 64,), jnp.float32),
        mesh=plsc.VectorSubcoreMesh(
            core_axis_name="c", subcore_axis_name="s",
            num_cores=NC, num_subcores=NS),
        scratch_types=[
            pltpu.VMEM((C,), jnp.int32),
            pltpu.VMEM((C,), jnp.int32),
            pltpu.VMEM((C, 256), jnp.float32),
            pltpu.VMEM((C, 256), jnp.float32),
            pltpu.VMEM((C, 64), jnp.float32),
            pltpu.VMEM((C, 64), jnp.float32),
            pltpu.VMEM((C, 64), jnp.float32),
            pltpu.VMEM((C,), jnp.float32),
            pltpu.VMEM((C,), jnp.float32),
            pltpu.VMEM((125, 64), jnp.float32),
            pltpu.VMEM((125, 64), jnp.float32),
            pltpu.VMEM((125 * 64,), jnp.float32),
            pltpu.VMEM_SHARED((N, 64), jnp.float32),
            pltpu.SemaphoreType.DMA,
            pltpu.SemaphoreType.DMA,
        ],
        compiler_params=pltpu.CompilerParams(
            use_tc_tiling_on_sc=False, needs_layout_passes=False),
    )


def _combine_body(p_ref, o_ref):
    o_ref[...] = p_ref[0] + p_ref[1]


@functools.lru_cache(maxsize=None)
def _make_combine(R):
    br = R // 5
    return pl.pallas_call(
        _combine_body,
        out_shape=jax.ShapeDtypeStruct((R, 128), jnp.float32),
        grid=(R // br,),
        in_specs=[pl.BlockSpec((2, br, 128), lambda i: (0, i, 0))],
        out_specs=pl.BlockSpec((br, 128), lambda i: (i, 0)),
    )


@jax.jit
def kernel(x, v, edges, omega_params):
    N, D = x.shape
    E = edges.shape[0]
    K = omega_params.shape[1]
    assert D == 128 and K == 8
    assert E % (NW * C) == 0 and N % (NS * 5) == 0

    u = jnp.concatenate([x, v], axis=1)
    src = edges[:, 0]
    dst = edges[:, 1]
    w = omega_params.reshape(E, K * K)

    partial = _make_sc(N, E)(u, src, dst, w)
    r = (N * K * K) // 128
    summed = _make_combine(r)(partial.reshape(2, r, 128))
    return summed.reshape(N, K, K)


# speedup vs baseline: 24.2684x; 1.2212x over previous
import functools

import jax
import jax.numpy as jnp
from jax import lax
from jax.experimental import pallas as pl
from jax.experimental.pallas import tpu as pltpu
from jax.experimental.pallas import tpu_sc as plsc

NC = 2
NS = 16
NW = NC * NS
C = 64
RB = 25


def _sc_body(N, E, u_hbm, src_hbm, dst_hbm, w_hbm, out_hbm,
             idx_s0, idx_t0, idx_s1, idx_t1, us0, ut0, us1, ut1,
             wv0, wv1, ws, wt, cs_arr, ct_arr,
             rowbuf, ov, b_sh, sem_g0, sem_g1, sem_w0, sem_w1):
    nchunk_tot = E // C
    main = nchunk_tot // NW
    extra = nchunk_tot % NW
    npairs = main // 2
    rpt = N // NS
    nbatch = rpt // RB
    cid = lax.axis_index("c")
    sid = lax.axis_index("s")
    wid = sid * NC + cid
    iota16 = lax.iota(jnp.int32, 16)
    zf = jnp.zeros((16,), jnp.float32)

    idx = ((idx_s0, idx_t0), (idx_s1, idx_t1))
    us = (us0, us1)
    ut = (ut0, ut1)
    wv = (wv0, wv1)
    sem_g = (sem_g0, sem_g1)
    sem_w = (sem_w0, sem_w1)

    def issue(b, i):
        base = jnp.where(i < main, (wid * main + i) * C,
                         jnp.minimum((main * NW + wid) * C, E - C))
        pltpu.sync_copy(src_hbm.at[pl.ds(base, C)], idx[b][0])
        pltpu.sync_copy(dst_hbm.at[pl.ds(base, C)], idx[b][1])
        pltpu.async_copy(u_hbm.at[idx[b][0]], us[b], sem_g[b])
        pltpu.async_copy(u_hbm.at[idx[b][1]], ut[b], sem_g[b])
        pltpu.async_copy(w_hbm.at[pl.ds(base, C)], wv[b], sem_w[b])

    def wait_gathers(b):
        pltpu.make_async_copy(u_hbm.at[idx[b][0]], us[b], sem_g[b]).wait()
        pltpu.make_async_copy(u_hbm.at[idx[b][1]], ut[b], sem_g[b]).wait()

    def wait_w(b):
        pltpu.make_async_copy(w_hbm.at[pl.ds(0, C)], wv[b], sem_w[b]).wait()

    def compute(b, flag=None):
        wait_gathers(b)

        def group_body(gi, gcarry):
            rows = gi * 16 + iota16

            def octet(o, accs):
                a0, a1, b0, b1, c0, c1 = accs
                cb = jnp.full((16,), o * 8, jnp.int32)
                for jj in range(8):
                    cx = cb + jj
                    cv = cx + 128
                    xs = plsc.load_gather(us[b], [rows, cx])
                    xt = plsc.load_gather(ut[b], [rows, cx])
                    vs = plsc.load_gather(us[b], [rows, cv])
                    vt = plsc.load_gather(ut[b], [rows, cv])
                    dd = xt - xs
                    if jj % 2 == 0:
                        a0 = a0 + dd * dd
                        b0 = b0 + vs * dd
                        c0 = c0 + vt * dd
                    else:
                        a1 = a1 + dd * dd
                        b1 = b1 + vs * dd
                        c1 = c1 + vt * dd
                return (a0, a1, b0, b1, c0, c1)

            a0, a1, b0, b1, c0, c1 = lax.fori_loop(
                0, 16, octet, (zf, zf, zf, zf, zf, zf))
            inv = 1.0 / jnp.maximum(a0 + a1, 1e-6)
            if flag is not None:
                inv = inv * flag
            cs_arr[pl.ds(gi * 16, 16)] = (b0 + b1) * inv
            ct_arr[pl.ds(gi * 16, 16)] = (c0 + c1) * inv
            return gcarry

        lax.fori_loop(0, C // 16, group_body, 0)
        wait_w(b)

        def scale2(e2, ecarry):
            for h in range(2):
                e = e2 * 2 + h
                esplat = jnp.full((16,), e, jnp.int32)
                csb = plsc.load_gather(cs_arr, [esplat])
                ctb = plsc.load_gather(ct_arr, [esplat])
                for g in range(4):
                    wvec = wv[b][e, pl.ds(g * 16, 16)]
                    ws[e, pl.ds(g * 16, 16)] = wvec * csb
                    wt[e, pl.ds(g * 16, 16)] = wvec * ctb
            return ecarry

        lax.fori_loop(0, C // 2, scale2, 0)
        pltpu.sync_copy(ws, b_sh.at[idx[b][0]], add=True)
        pltpu.sync_copy(wt, b_sh.at[idx[b][1]], add=True)

    issue(0, 0)
    zero16 = jnp.zeros((16,), jnp.float32)

    def zrow(r, carry):
        for g in range(4):
            rowbuf[r, pl.ds(g * 16, 16)] = zero16
        return carry

    lax.fori_loop(0, RB, zrow, 0)
    row0 = sid * rpt

    def zbatch(j, carry):
        pltpu.sync_copy(rowbuf, b_sh.at[pl.ds(row0 + j * RB, RB)])
        return carry

    lax.fori_loop(0, nbatch, zbatch, 0)
    plsc.subcore_barrier()

    issue(1, 1)

    def pair(j, carry):
        compute(0)
        issue(0, 2 * j + 2)
        compute(1)
        issue(1, 2 * j + 3)
        return carry

    lax.fori_loop(0, npairs, pair, 0)
    compute(0, flag=jnp.where(wid < extra, 1.0, 0.0).astype(jnp.float32))
    wait_gathers(1)
    wait_w(1)
    plsc.subcore_barrier()

    perms = []
    for g in range(4):
        fl = iota16 + (g * 16)
        i8 = fl // 8
        j8 = fl % 8
        perms.append(j8 * 8 + i8)

    def abatch(j, carry):
        pltpu.sync_copy(b_sh.at[pl.ds(row0 + j * RB, RB)], rowbuf)

        def arow(r, rcarry):
            rsplat = jnp.full((16,), r, jnp.int32)
            for g in range(4):
                bvec = rowbuf[r, pl.ds(g * 16, 16)]
                btv = plsc.load_gather(rowbuf, [rsplat, perms[g]])
                ov[pl.ds(r * 64 + g * 16, 16)] = 0.5 * (bvec - btv)
            return rcarry

        lax.fori_loop(0, RB, arow, 0)
        pltpu.sync_copy(
            ov,
            out_hbm.at[pl.ds((cid * N + row0 + j * RB) * 64, RB * 64)])
        return carry

    lax.fori_loop(0, nbatch, abatch, 0)


@functools.lru_cache(maxsize=None)
def _make_sc(N, E):
    body = functools.partial(_sc_body, N, E)
    return pl.kernel(
        body,
        out_type=jax.ShapeDtypeStruct((NC * N * 64,), jnp.float32),
        mesh=plsc.VectorSubcoreMesh(
            core_axis_name="c", subcore_axis_name="s",
            num_cores=NC, num_subcores=NS),
        scratch_types=[
            pltpu.VMEM((C,), jnp.int32),
            pltpu.VMEM((C,), jnp.int32),
            pltpu.VMEM((C,), jnp.int32),
            pltpu.VMEM((C,), jnp.int32),
            pltpu.VMEM((C, 256), jnp.float32),
            pltpu.VMEM((C, 256), jnp.float32),
            pltpu.VMEM((C, 256), jnp.float32),
            pltpu.VMEM((C, 256), jnp.float32),
            pltpu.VMEM((C, 64), jnp.float32),
            pltpu.VMEM((C, 64), jnp.float32),
            pltpu.VMEM((C, 64), jnp.float32),
            pltpu.VMEM((C, 64), jnp.float32),
            pltpu.VMEM((C,), jnp.float32),
            pltpu.VMEM((C,), jnp.float32),
            pltpu.VMEM((RB, 64), jnp.float32),
            pltpu.VMEM((RB * 64,), jnp.float32),
            pltpu.VMEM_SHARED((N, 64), jnp.float32),
            pltpu.SemaphoreType.DMA,
            pltpu.SemaphoreType.DMA,
            pltpu.SemaphoreType.DMA,
            pltpu.SemaphoreType.DMA,
        ],
        compiler_params=pltpu.CompilerParams(
            use_tc_tiling_on_sc=False, needs_layout_passes=False),
    )


def _combine_body(p_ref, o_ref):
    o_ref[...] = p_ref[0] + p_ref[1]


@functools.lru_cache(maxsize=None)
def _make_combine(R):
    br = R // 5
    return pl.pallas_call(
        _combine_body,
        out_shape=jax.ShapeDtypeStruct((R, 128), jnp.float32),
        grid=(R // br,),
        in_specs=[pl.BlockSpec((2, br, 128), lambda i: (0, i, 0))],
        out_specs=pl.BlockSpec((br, 128), lambda i: (i, 0)),
    )


@jax.jit
def kernel(x, v, edges, omega_params):
    N, D = x.shape
    E = edges.shape[0]
    K = omega_params.shape[1]
    assert D == 128 and K == 8
    assert E % C == 0 and N % (NS * RB) == 0
    assert ((E // C) // NW) % 2 == 0

    u = jnp.concatenate([x, v], axis=1)
    src = edges[:, 0]
    dst = edges[:, 1]
    w = omega_params.reshape(E, K * K)

    partial = _make_sc(N, E)(u, src, dst, w)
    r = (N * K * K) // 128
    summed = _make_combine(r)(partial.reshape(2, r, 128))
    return summed.reshape(N, K, K)


# speedup vs baseline: 78.6070x; 3.2391x over previous
import functools

import jax
import jax.numpy as jnp
from jax import lax
from jax.experimental import pallas as pl
from jax.experimental.pallas import tpu as pltpu
from jax.experimental.pallas import tpu_sc as plsc

NC = 2
NS = 16
NW = NC * NS
C = 64
RB = 25


def _sc_body(N, E, u_hbm, src_hbm, dst_hbm, w_hbm, out_hbm,
             idx_s0, idx_t0, idx_s1, idx_t1, us0, ut0, us1, ut1,
             wv0, wv1, ws, wt, cs_arr, ct_arr,
             rowbuf, ov, b_sh, sem_g0, sem_g1, sem_w0, sem_w1):
    nchunk_tot = E // C
    main = nchunk_tot // NW
    extra = nchunk_tot % NW
    npairs = main // 2
    rpt = N // NS
    nbatch = rpt // RB
    cid = lax.axis_index("c")
    sid = lax.axis_index("s")
    wid = sid * NC + cid
    iota16 = lax.iota(jnp.int32, 16)
    zf = jnp.zeros((16,), jnp.float32)

    idx = ((idx_s0, idx_t0), (idx_s1, idx_t1))
    us = (us0, us1)
    ut = (ut0, ut1)
    wv = (wv0, wv1)
    sem_g = (sem_g0, sem_g1)
    sem_w = (sem_w0, sem_w1)

    def issue(b, i):
        base = jnp.where(i < main, (wid * main + i) * C,
                         jnp.minimum((main * NW + wid) * C, E - C))
        pltpu.sync_copy(src_hbm.at[pl.ds(base, C)], idx[b][0])
        pltpu.sync_copy(dst_hbm.at[pl.ds(base, C)], idx[b][1])
        pltpu.async_copy(u_hbm.at[idx[b][0]], us[b], sem_g[b])
        pltpu.async_copy(u_hbm.at[idx[b][1]], ut[b], sem_g[b])
        pltpu.async_copy(w_hbm.at[pl.ds(base, C)], wv[b], sem_w[b])

    def wait_gathers(b):
        pltpu.make_async_copy(u_hbm.at[idx[b][0]], us[b], sem_g[b]).wait()
        pltpu.make_async_copy(u_hbm.at[idx[b][1]], ut[b], sem_g[b]).wait()

    def wait_w(b):
        pltpu.make_async_copy(w_hbm.at[pl.ds(0, C)], wv[b], sem_w[b]).wait()

    def compute(b, flag=None):
        wait_gathers(b)

        rot = (iota16 * 17) & 127

        def group_body(gi, gcarry):
            rows = gi * 16 + iota16

            def octet(o, accs):
                a0, a1, b0, b1, c0, c1 = accs
                cb = jnp.full((16,), o * 8, jnp.int32) + rot
                for jj in range(8):
                    cx = (cb + jj) & 127
                    cv = cx + 128
                    xs = plsc.load_gather(us[b], [rows, cx])
                    xt = plsc.load_gather(ut[b], [rows, cx])
                    vs = plsc.load_gather(us[b], [rows, cv])
                    vt = plsc.load_gather(ut[b], [rows, cv])
                    dd = xt - xs
                    if jj % 2 == 0:
                        a0 = a0 + dd * dd
                        b0 = b0 + vs * dd
                        c0 = c0 + vt * dd
                    else:
                        a1 = a1 + dd * dd
                        b1 = b1 + vs * dd
                        c1 = c1 + vt * dd
                return (a0, a1, b0, b1, c0, c1)

            a0, a1, b0, b1, c0, c1 = lax.fori_loop(
                0, 16, octet, (zf, zf, zf, zf, zf, zf))
            inv = 1.0 / jnp.maximum(a0 + a1, 1e-6)
            if flag is not None:
                inv = inv * flag
            cs_arr[pl.ds(gi * 16, 16)] = (b0 + b1) * inv
            ct_arr[pl.ds(gi * 16, 16)] = (c0 + c1) * inv
            return gcarry

        lax.fori_loop(0, C // 16, group_body, 0)
        wait_w(b)

        def scale2(e2, ecarry):
            for h in range(2):
                e = e2 * 2 + h
                esplat = jnp.full((16,), e, jnp.int32)
                csb = plsc.load_gather(cs_arr, [esplat])
                ctb = plsc.load_gather(ct_arr, [esplat])
                for g in range(4):
                    wvec = wv[b][e, pl.ds(g * 16, 16)]
                    ws[e, pl.ds(g * 16, 16)] = wvec * csb
                    wt[e, pl.ds(g * 16, 16)] = wvec * ctb
            return ecarry

        lax.fori_loop(0, C // 2, scale2, 0)
        pltpu.sync_copy(ws, b_sh.at[idx[b][0]], add=True)
        pltpu.sync_copy(wt, b_sh.at[idx[b][1]], add=True)

    issue(0, 0)
    zero16 = jnp.zeros((16,), jnp.float32)

    def zrow(r, carry):
        for g in range(4):
            rowbuf[r, pl.ds(g * 16, 16)] = zero16
        return carry

    lax.fori_loop(0, RB, zrow, 0)
    row0 = sid * rpt

    def zbatch(j, carry):
        pltpu.sync_copy(rowbuf, b_sh.at[pl.ds(row0 + j * RB, RB)])
        return carry

    lax.fori_loop(0, nbatch, zbatch, 0)
    plsc.subcore_barrier()

    issue(1, 1)

    def pair(j, carry):
        compute(0)
        issue(0, 2 * j + 2)
        compute(1)
        issue(1, 2 * j + 3)
        return carry

    lax.fori_loop(0, npairs, pair, 0)
    compute(0, flag=jnp.where(wid < extra, 1.0, 0.0).astype(jnp.float32))
    wait_gathers(1)
    wait_w(1)
    plsc.subcore_barrier()

    perms = []
    for g in range(4):
        fl = iota16 + (g * 16)
        i8 = fl // 8
        j8 = fl % 8
        perms.append(j8 * 8 + i8)

    def abatch(j, carry):
        pltpu.sync_copy(b_sh.at[pl.ds(row0 + j * RB, RB)], rowbuf)

        def arow(r, rcarry):
            rsplat = jnp.full((16,), r, jnp.int32)
            for g in range(4):
                bvec = rowbuf[r, pl.ds(g * 16, 16)]
                btv = plsc.load_gather(rowbuf, [rsplat, perms[g]])
                ov[pl.ds(r * 64 + g * 16, 16)] = 0.5 * (bvec - btv)
            return rcarry

        lax.fori_loop(0, RB, arow, 0)
        pltpu.sync_copy(
            ov,
            out_hbm.at[pl.ds((cid * N + row0 + j * RB) * 64, RB * 64)])
        return carry

    lax.fori_loop(0, nbatch, abatch, 0)


@functools.lru_cache(maxsize=None)
def _make_sc(N, E):
    body = functools.partial(_sc_body, N, E)
    return pl.kernel(
        body,
        out_type=jax.ShapeDtypeStruct((NC * N * 64,), jnp.float32),
        mesh=plsc.VectorSubcoreMesh(
            core_axis_name="c", subcore_axis_name="s",
            num_cores=NC, num_subcores=NS),
        scratch_types=[
            pltpu.VMEM((C,), jnp.int32),
            pltpu.VMEM((C,), jnp.int32),
            pltpu.VMEM((C,), jnp.int32),
            pltpu.VMEM((C,), jnp.int32),
            pltpu.VMEM((C, 256), jnp.float32),
            pltpu.VMEM((C, 256), jnp.float32),
            pltpu.VMEM((C, 256), jnp.float32),
            pltpu.VMEM((C, 256), jnp.float32),
            pltpu.VMEM((C, 64), jnp.float32),
            pltpu.VMEM((C, 64), jnp.float32),
            pltpu.VMEM((C, 64), jnp.float32),
            pltpu.VMEM((C, 64), jnp.float32),
            pltpu.VMEM((C,), jnp.float32),
            pltpu.VMEM((C,), jnp.float32),
            pltpu.VMEM((RB, 64), jnp.float32),
            pltpu.VMEM((RB * 64,), jnp.float32),
            pltpu.VMEM_SHARED((N, 64), jnp.float32),
            pltpu.SemaphoreType.DMA,
            pltpu.SemaphoreType.DMA,
            pltpu.SemaphoreType.DMA,
            pltpu.SemaphoreType.DMA,
        ],
        compiler_params=pltpu.CompilerParams(
            use_tc_tiling_on_sc=False, needs_layout_passes=False),
    )


def _combine_body(p_ref, o_ref):
    o_ref[...] = p_ref[0] + p_ref[1]


@functools.lru_cache(maxsize=None)
def _make_combine(R):
    br = R // 5
    return pl.pallas_call(
        _combine_body,
        out_shape=jax.ShapeDtypeStruct((R, 128), jnp.float32),
        grid=(R // br,),
        in_specs=[pl.BlockSpec((2, br, 128), lambda i: (0, i, 0))],
        out_specs=pl.BlockSpec((br, 128), lambda i: (i, 0)),
    )


@jax.jit
def kernel(x, v, edges, omega_params):
    N, D = x.shape
    E = edges.shape[0]
    K = omega_params.shape[1]
    assert D == 128 and K == 8
    assert E % C == 0 and N % (NS * RB) == 0
    assert ((E // C) // NW) % 2 == 0

    u = jnp.concatenate([x, v], axis=1)
    src = edges[:, 0]
    dst = edges[:, 1]
    w = omega_params.reshape(E, K * K)

    partial = _make_sc(N, E)(u, src, dst, w)
    r = (N * K * K) // 128
    summed = _make_combine(r)(partial.reshape(2, r, 128))
    return summed.reshape(N, K, K)
